# baseline (speedup 1.0000x reference)
import jax
import jax.numpy as jnp
from jax.experimental import pallas as pl

T = 500
DROPOUT_P = 0.1
TPAD = 512
R = 64


def _ddpm_body(x_ref, n_ref, ts_ref, u_ref, ta_ref, tb_ref,
               xt_ref, nout_ref, tn_ref, cm_ref):
    ts = ts_ref[...]
    lane = jax.lax.broadcasted_iota(jnp.int32, (R, TPAD), 1)
    onehot = lane == ts
    a = jnp.sum(jnp.where(onehot, ta_ref[...], 0.0), axis=1, keepdims=True)
    b = jnp.sum(jnp.where(onehot, tb_ref[...], 0.0), axis=1, keepdims=True)
    n = n_ref[...]
    xt_ref[...] = a * x_ref[...] + b * n
    nout_ref[...] = n
    tn_ref[...] = ts.astype(jnp.float32) / T
    cm_ref[...] = (u_ref[...] < DROPOUT_P).astype(jnp.float32)


def kernel(x, cls, timestep, noise, u, sqrt_abar_t, sqrt_abar_t1):
    B = x.shape[0]
    F = x.shape[1] * x.shape[2] * x.shape[3]
    x2 = x.reshape(B, F)
    n2 = noise.reshape(B, F)
    ts2 = timestep.reshape(B, 1)
    u2 = u.reshape(B, 1)
    ta = jnp.zeros((1, TPAD), jnp.float32).at[0, :T].set(sqrt_abar_t)
    tb = jnp.zeros((1, TPAD), jnp.float32).at[0, :T].set(sqrt_abar_t1)

    grid = (B // R,)
    big = pl.BlockSpec((R, F), lambda i: (i, 0))
    col = pl.BlockSpec((R, 1), lambda i: (i, 0))
    tab = pl.BlockSpec((1, TPAD), lambda i: (0, 0))

    xt2, nout2, tn2, cm2 = pl.pallas_call(
        _ddpm_body,
        grid=grid,
        in_specs=[big, big, col, col, tab, tab],
        out_specs=[big, big, col, col],
        out_shape=[
            jax.ShapeDtypeStruct((B, F), jnp.float32),
            jax.ShapeDtypeStruct((B, F), jnp.float32),
            jax.ShapeDtypeStruct((B, 1), jnp.float32),
            jax.ShapeDtypeStruct((B, 1), jnp.float32),
        ],
    )(x2, n2, ts2, u2, ta, tb)

    return (nout2.reshape(x.shape), xt2.reshape(x.shape), cls,
            tn2.reshape(B), cm2.reshape(B))
